# baseline (speedup 1.0000x reference)
import functools

import jax
import jax.numpy as jnp
from jax import lax
from jax.experimental import pallas as pl
from jax.experimental.pallas import tpu as pltpu
from jax.experimental.pallas import tpu_sc as plsc

_S = 32


def kernel(padded_input, lengths):
    B, L, D = padded_input.shape
    x2d = padded_input.reshape(B * L, D)
    lens32 = lengths.astype(jnp.int32)

    info = plsc.get_sparse_core_info()
    NC, NS = info.num_cores, info.num_subcores
    assert NC * NS == _S and B == 16

    mesh = plsc.VectorSubcoreMesh(core_axis_name="c", subcore_axis_name="s")

    @functools.partial(
        pl.kernel,
        mesh=mesh,
        out_type=jax.ShapeDtypeStruct((B, _S * D + 1), jnp.float32),
        scratch_types=[
            pltpu.VMEM((B,), jnp.int32),
            pltpu.VMEM((B,), jnp.int32),
            pltpu.VMEM((B,), jnp.int32),
            pltpu.VMEM((B, D), jnp.float32),
            pltpu.VMEM((B, D), jnp.float32),
            pltpu.VMEM((B, D), jnp.float32),
            pltpu.VMEM((B, 1), jnp.float32),
            pltpu.SemaphoreType.DMA,
            pltpu.SemaphoreType.DMA,
        ],
        compiler_params=pltpu.CompilerParams(
            needs_layout_passes=False,
            skip_device_barrier=True,
            disable_bounds_checks=True,
            disable_semaphore_checks=True,
        ),
    )
    def run(x_hbm, len_hbm, out_hbm, lens_v, idx_lo_v, idx_hi_v,
            rows_lo_v, rows_hi_v, out_v, col_v, sem_lo, sem_hi):
        k = lax.axis_index("s") * NC + lax.axis_index("c")

        pltpu.sync_copy(len_hbm, lens_v)
        lens = lens_v[...]

        g = jnp.minimum(lens & (-lens), _S)
        step = lens // g
        j = jnp.broadcast_to(k, (B,)).astype(jnp.int32) * step
        scale = g.astype(jnp.float32) * (1.0 / _S)
        pos = (j.astype(jnp.float32) + 0.5) * scale - 0.5
        pos = jnp.clip(pos, 0.0, (lens - 1).astype(jnp.float32))
        lo = pos.astype(jnp.int32)
        hi = jnp.minimum(lo + 1, lens - 1)
        w = pos - lo.astype(jnp.float32)

        def vgather(vec, idx):
            return lax.gather(
                vec, idx.reshape(16, 1).astype(jnp.int32),
                dimension_numbers=lax.GatherDimensionNumbers(
                    offset_dims=(), collapsed_slice_dims=(0,),
                    start_index_map=(0,)),
                slice_sizes=(1,),
                mode=lax.GatherScatterMode.PROMISE_IN_BOUNDS)

        biota = lax.iota(jnp.int32, B)
        glo = biota * L + lo
        ghi = biota * L + hi
        half = B // 2
        first = biota < half
        idx_lo_v[...] = jnp.where(first, glo, vgather(ghi, (biota - half) & 15))
        idx_hi_v[...] = jnp.where(first, vgather(glo, (biota + half) & 15), ghi)
        copy_a = pltpu.async_copy(x_hbm.at[idx_lo_v], rows_lo_v, sem_lo)
        copy_b = pltpu.async_copy(x_hbm.at[idx_hi_v], rows_hi_v, sem_hi)

        copy_a.wait()

        @plsc.parallel_loop(0, half, unroll=2)
        def _lerp_a(b):
            wb = vgather(w, jnp.broadcast_to(b, (16,)))
            for c in range(D // 16):
                sl = pl.ds(c * 16, 16)
                lo_ch = rows_lo_v[b, sl]
                hi_ch = rows_lo_v[half + b, sl]
                out_v[b, sl] = lo_ch + wb * (hi_ch - lo_ch)

        out_a = pltpu.async_copy(
            out_v.at[pl.ds(0, half)],
            out_hbm.at[pl.ds(0, half), pl.ds(k * D, D)], sem_lo)
        copy_b.wait()

        @plsc.parallel_loop(0, half, unroll=2)
        def _lerp_b(b):
            wb = vgather(w, jnp.broadcast_to(half + b, (16,)))
            for c in range(D // 16):
                sl = pl.ds(c * 16, 16)
                lo_ch = rows_hi_v[b, sl]
                hi_ch = rows_hi_v[half + b, sl]
                out_v[half + b, sl] = lo_ch + wb * (hi_ch - lo_ch)

        out_b = pltpu.async_copy(
            out_v.at[pl.ds(half, half)],
            out_hbm.at[pl.ds(half, half), pl.ds(k * D, D)], sem_hi)
        out_a.wait()
        out_b.wait()

        @pl.when(k == 0)
        def _():
            plsc.store_scatter(
                col_v, [biota, jnp.zeros((16,), jnp.int32)],
                lens.astype(jnp.float32))
            pltpu.sync_copy(col_v, out_hbm.at[:, pl.ds(_S * D, 1)])

    return run(x2d, lens32)


# speedup vs baseline: 1.0524x; 1.0524x over previous
import functools

import jax
import jax.numpy as jnp
from jax import lax
from jax.experimental import pallas as pl
from jax.experimental.pallas import tpu as pltpu
from jax.experimental.pallas import tpu_sc as plsc

_S = 32


def kernel(padded_input, lengths):
    B, L, D = padded_input.shape
    x2d = padded_input.reshape(B * L, D)
    lens32 = lengths.astype(jnp.int32)

    info = plsc.get_sparse_core_info()
    NC, NS = info.num_cores, info.num_subcores
    assert NC * NS == _S and B == 16

    mesh = plsc.VectorSubcoreMesh(core_axis_name="c", subcore_axis_name="s")

    @functools.partial(
        pl.kernel,
        mesh=mesh,
        out_type=jax.ShapeDtypeStruct((B, _S * D + 1), jnp.float32),
        scratch_types=[
            pltpu.VMEM((B,), jnp.int32),
            pltpu.VMEM((B,), jnp.int32),
            pltpu.VMEM((B,), jnp.int32),
            pltpu.VMEM((B, D), jnp.float32),
            pltpu.VMEM((B, D), jnp.float32),
            pltpu.VMEM((B, D), jnp.float32),
            pltpu.VMEM((B, 1), jnp.float32),
            pltpu.SemaphoreType.DMA,
            pltpu.SemaphoreType.DMA,
        ],
        compiler_params=pltpu.CompilerParams(
            needs_layout_passes=False,
            skip_device_barrier=True,
            disable_bounds_checks=True,
            disable_semaphore_checks=True,
        ),
    )
    def run(x_hbm, len_hbm, out_hbm, lens_v, idx_lo_v, idx_hi_v,
            rows_lo_v, rows_hi_v, out_v, col_v, sem_lo, sem_hi):
        k = lax.axis_index("s") * NC + lax.axis_index("c")

        pltpu.sync_copy(len_hbm, lens_v)
        lens = lens_v[...]

        g = jnp.minimum(lens & (-lens), _S)
        step = lens // g
        j = jnp.broadcast_to(k, (B,)).astype(jnp.int32) * step
        scale = g.astype(jnp.float32) * (1.0 / _S)
        pos = (j.astype(jnp.float32) + 0.5) * scale - 0.5
        pos = jnp.clip(pos, 0.0, (lens - 1).astype(jnp.float32))
        lo = pos.astype(jnp.int32)
        hi = jnp.minimum(lo + 1, lens - 1)
        w = pos - lo.astype(jnp.float32)

        def vgather(vec, idx):
            return lax.gather(
                vec, idx.reshape(16, 1).astype(jnp.int32),
                dimension_numbers=lax.GatherDimensionNumbers(
                    offset_dims=(), collapsed_slice_dims=(0,),
                    start_index_map=(0,)),
                slice_sizes=(1,),
                mode=lax.GatherScatterMode.PROMISE_IN_BOUNDS)

        biota = lax.iota(jnp.int32, B)
        glo = biota * L + lo
        ghi = biota * L + hi
        half = B // 2
        first = biota < half
        idx_lo_v[...] = jnp.where(first, glo, vgather(ghi, (biota - half) & 15))
        idx_hi_v[...] = jnp.where(first, vgather(glo, (biota + half) & 15), ghi)
        copy_a = pltpu.async_copy(x_hbm.at[idx_lo_v], rows_lo_v, sem_lo)
        copy_b = pltpu.async_copy(x_hbm.at[idx_hi_v], rows_hi_v, sem_hi)

        copy_a.wait()

        @plsc.parallel_loop(0, half * (D // 16), unroll=16)
        def _lerp_a(i):
            b = i >> 4
            sl = pl.ds((i & 15) * 16, 16)
            lo_ch = rows_lo_v[b, sl]
            hi_ch = rows_lo_v[half + b, sl]
            out_v[b, sl] = lo_ch + vgather(w, jnp.broadcast_to(b, (16,))) * (
                hi_ch - lo_ch)

        out_a = pltpu.async_copy(
            out_v.at[pl.ds(0, half)],
            out_hbm.at[pl.ds(0, half), pl.ds(k * D, D)], sem_lo)
        copy_b.wait()

        @plsc.parallel_loop(0, half * (D // 16), unroll=16)
        def _lerp_b(i):
            b = i >> 4
            sl = pl.ds((i & 15) * 16, 16)
            lo_ch = rows_hi_v[b, sl]
            hi_ch = rows_hi_v[half + b, sl]
            out_v[half + b, sl] = lo_ch + vgather(
                w, jnp.broadcast_to(half + b, (16,))) * (hi_ch - lo_ch)

        out_b = pltpu.async_copy(
            out_v.at[pl.ds(half, half)],
            out_hbm.at[pl.ds(half, half), pl.ds(k * D, D)], sem_hi)
        out_a.wait()
        out_b.wait()

        @pl.when(k == 0)
        def _():
            plsc.store_scatter(
                col_v, [biota, jnp.zeros((16,), jnp.int32)],
                lens.astype(jnp.float32))
            pltpu.sync_copy(col_v, out_hbm.at[:, pl.ds(_S * D, 1)])

    return run(x2d, lens32)


# speedup vs baseline: 1.0754x; 1.0219x over previous
import functools

import jax
import jax.numpy as jnp
from jax import lax
from jax.experimental import pallas as pl
from jax.experimental.pallas import tpu as pltpu
from jax.experimental.pallas import tpu_sc as plsc

_S = 32


def kernel(padded_input, lengths):
    B, L, D = padded_input.shape
    x2d = padded_input.reshape(B * L, D)
    lens32 = lengths.astype(jnp.int32)

    info = plsc.get_sparse_core_info()
    NC, NS = info.num_cores, info.num_subcores
    assert NC * NS == _S and B == 16

    mesh = plsc.VectorSubcoreMesh(core_axis_name="c", subcore_axis_name="s")

    @functools.partial(
        pl.kernel,
        mesh=mesh,
        out_type=jax.ShapeDtypeStruct((B, _S * D + 1), jnp.float32),
        scratch_types=[
            pltpu.VMEM((B,), jnp.int32),
            pltpu.VMEM((B,), jnp.int32),
            pltpu.VMEM((B,), jnp.int32),
            pltpu.VMEM((B, D), jnp.float32),
            pltpu.VMEM((B, D), jnp.float32),
            pltpu.VMEM((B, D), jnp.float32),
            pltpu.VMEM((B, 1), jnp.float32),
            pltpu.SemaphoreType.DMA,
            pltpu.SemaphoreType.DMA,
        ],
        compiler_params=pltpu.CompilerParams(
            needs_layout_passes=False,
            skip_device_barrier=True,
            disable_bounds_checks=True,
            disable_semaphore_checks=True,
        ),
    )
    def run(x_hbm, len_hbm, out_hbm, lens_v, idx_lo_v, idx_hi_v,
            rows_lo_v, rows_hi_v, out_v, col_v, sem_lo, sem_hi):
        k = lax.axis_index("s") * NC + lax.axis_index("c")

        pltpu.sync_copy(len_hbm, lens_v)
        lens = lens_v[...]

        g = jnp.minimum(lens & (-lens), _S)
        step = lens // g
        j = jnp.broadcast_to(k, (B,)).astype(jnp.int32) * step
        scale = g.astype(jnp.float32) * (1.0 / _S)
        pos = (j.astype(jnp.float32) + 0.5) * scale - 0.5
        pos = jnp.clip(pos, 0.0, (lens - 1).astype(jnp.float32))
        lo = pos.astype(jnp.int32)
        hi = jnp.minimum(lo + 1, lens - 1)
        w = pos - lo.astype(jnp.float32)

        def vgather(vec, idx):
            return lax.gather(
                vec, idx.reshape(16, 1).astype(jnp.int32),
                dimension_numbers=lax.GatherDimensionNumbers(
                    offset_dims=(), collapsed_slice_dims=(0,),
                    start_index_map=(0,)),
                slice_sizes=(1,),
                mode=lax.GatherScatterMode.PROMISE_IN_BOUNDS)

        biota = lax.iota(jnp.int32, B)
        glo = biota * L + lo
        ghi = biota * L + hi
        half = B // 2
        first = biota < half
        idx_lo_v[...] = jnp.where(first, glo, vgather(ghi, (biota - half) & 15))
        idx_hi_v[...] = jnp.where(first, vgather(glo, (biota + half) & 15), ghi)
        copy_a = pltpu.async_copy(x_hbm.at[idx_lo_v], rows_lo_v, sem_lo)
        copy_b = pltpu.async_copy(x_hbm.at[idx_hi_v], rows_hi_v, sem_hi)

        @pl.when(k == 0)
        def _():
            plsc.store_scatter(
                col_v, [biota, jnp.zeros((16,), jnp.int32)],
                lens.astype(jnp.float32))
            pltpu.sync_copy(col_v, out_hbm.at[:, pl.ds(_S * D, 1)])

        copy_a.wait()

        @plsc.parallel_loop(0, half * (D // 16), unroll=8)
        def _lerp_a(i):
            b = i >> 4
            sl = pl.ds((i & 15) * 16, 16)
            lo_ch = rows_lo_v[b, sl]
            hi_ch = rows_lo_v[half + b, sl]
            out_v[b, sl] = lo_ch + vgather(w, jnp.broadcast_to(b, (16,))) * (
                hi_ch - lo_ch)

        out_a = pltpu.async_copy(
            out_v.at[pl.ds(0, half)],
            out_hbm.at[pl.ds(0, half), pl.ds(k * D, D)], sem_lo)
        copy_b.wait()

        @plsc.parallel_loop(0, half * (D // 16), unroll=8)
        def _lerp_b(i):
            b = i >> 4
            sl = pl.ds((i & 15) * 16, 16)
            lo_ch = rows_hi_v[b, sl]
            hi_ch = rows_hi_v[half + b, sl]
            out_v[half + b, sl] = lo_ch + vgather(
                w, jnp.broadcast_to(half + b, (16,))) * (hi_ch - lo_ch)

        out_b = pltpu.async_copy(
            out_v.at[pl.ds(half, half)],
            out_hbm.at[pl.ds(half, half), pl.ds(k * D, D)], sem_hi)
        out_a.wait()
        out_b.wait()

    return run(x2d, lens32)


# speedup vs baseline: 1.0822x; 1.0063x over previous
import functools

import jax
import jax.numpy as jnp
from jax import lax
from jax.experimental import pallas as pl
from jax.experimental.pallas import tpu as pltpu
from jax.experimental.pallas import tpu_sc as plsc

_S = 32


def kernel(padded_input, lengths):
    B, L, D = padded_input.shape
    x2d = padded_input.reshape(B * L, D)
    lens32 = lengths.astype(jnp.int32)

    info = plsc.get_sparse_core_info()
    NC, NS = info.num_cores, info.num_subcores
    assert NC * NS == _S and B == 16 and NC == 2

    half = _S // 2
    quarter = half // 2

    mesh = plsc.VectorSubcoreMesh(core_axis_name="c", subcore_axis_name="s")

    @functools.partial(
        pl.kernel,
        mesh=mesh,
        out_type=jax.ShapeDtypeStruct((B, _S * D + 1), jnp.float32),
        scratch_types=[
            pltpu.VMEM((B,), jnp.int32),
            pltpu.VMEM((16,), jnp.int32),
            pltpu.VMEM((16,), jnp.int32),
            pltpu.VMEM((16, D), jnp.float32),
            pltpu.VMEM((16, D), jnp.float32),
            pltpu.VMEM((half * D,), jnp.float32),
            pltpu.VMEM((B, 1), jnp.float32),
            pltpu.SemaphoreType.DMA,
            pltpu.SemaphoreType.DMA,
        ],
        compiler_params=pltpu.CompilerParams(
            needs_layout_passes=False,
            skip_device_barrier=True,
            disable_bounds_checks=True,
            disable_semaphore_checks=True,
        ),
    )
    def run(x_hbm, len_hbm, out_hbm, lens_v, idx_a_v, idx_b_v,
            rows_a_v, rows_b_v, out_v, col_v, sem_a, sem_b):
        r = lax.axis_index("s")
        h = lax.axis_index("c")

        pltpu.sync_copy(len_hbm, lens_v)
        lens = lens_v[...]

        def vgather(vec, idx):
            return lax.gather(
                vec, idx.reshape(16, 1).astype(jnp.int32),
                dimension_numbers=lax.GatherDimensionNumbers(
                    offset_dims=(), collapsed_slice_dims=(0,),
                    start_index_map=(0,)),
                slice_sizes=(1,),
                mode=lax.GatherScatterMode.PROMISE_IN_BOUNDS)

        def splat(x):
            return jnp.broadcast_to(x, (16,)).astype(jnp.int32)

        len_b = vgather(lens, splat(r))
        g = jnp.minimum(len_b & (-len_b), _S)
        step = len_b // g
        biota = lax.iota(jnp.int32, 16)
        kvec = biota + splat(h * half)
        j = kvec * step
        scale = g.astype(jnp.float32) * (1.0 / _S)
        pos = (j.astype(jnp.float32) + 0.5) * scale - 0.5
        pos = jnp.clip(pos, 0.0, (len_b - 1).astype(jnp.float32))
        lo = pos.astype(jnp.int32)
        hi = jnp.minimum(lo + 1, len_b - 1)
        w = pos - lo.astype(jnp.float32)

        glo = splat(r * L) + lo
        ghi = splat(r * L) + hi
        first = biota < quarter
        idx_a_v[...] = jnp.where(first, glo, vgather(ghi, (biota - quarter) & 15))
        idx_b_v[...] = jnp.where(first, vgather(glo, (biota + quarter) & 15), ghi)
        copy_a = pltpu.async_copy(x_hbm.at[idx_a_v], rows_a_v, sem_a)
        copy_b = pltpu.async_copy(x_hbm.at[idx_b_v], rows_b_v, sem_b)

        @pl.when((r == 0) & (h == 0))
        def _():
            plsc.store_scatter(
                col_v, [biota, jnp.zeros((16,), jnp.int32)],
                lens.astype(jnp.float32))
            pltpu.sync_copy(col_v, out_hbm.at[:, pl.ds(_S * D, 1)])

        cpq = quarter * (D // 16)

        copy_a.wait()

        @plsc.parallel_loop(0, cpq, unroll=8)
        def _lerp_a(i):
            s = i >> 4
            sl = pl.ds(i * 16, 16)
            lo_ch = rows_a_v[s, pl.ds((i & 15) * 16, 16)]
            hi_ch = rows_a_v[quarter + s, pl.ds((i & 15) * 16, 16)]
            out_v[sl] = lo_ch + vgather(w, splat(s)) * (hi_ch - lo_ch)

        out_a = pltpu.async_copy(
            out_v.at[pl.ds(0, quarter * D)],
            out_hbm.at[r, pl.ds(h * half * D, quarter * D)], sem_a)
        copy_b.wait()

        @plsc.parallel_loop(0, cpq, unroll=8)
        def _lerp_b(i):
            s = i >> 4
            sl = pl.ds(quarter * D + i * 16, 16)
            lo_ch = rows_b_v[s, pl.ds((i & 15) * 16, 16)]
            hi_ch = rows_b_v[quarter + s, pl.ds((i & 15) * 16, 16)]
            out_v[sl] = lo_ch + vgather(w, splat(quarter + s)) * (hi_ch - lo_ch)

        out_b = pltpu.async_copy(
            out_v.at[pl.ds(quarter * D, quarter * D)],
            out_hbm.at[r, pl.ds(h * half * D + quarter * D, quarter * D)],
            sem_b)
        out_a.wait()
        out_b.wait()

    return run(x2d, lens32)
